# baseline (speedup 1.0000x reference)
import jax
import jax.numpy as jnp
from jax.experimental import pallas as pl
from jax.experimental.pallas import tpu as pltpu

_P = 128


def _deinterleave(v):
    p, w = v.shape
    i = jax.lax.broadcasted_iota(jnp.int32, (p, 128), 1)
    perm = jnp.where(i < 64, 2 * i, 2 * i - 127)
    ev, od = [], []
    for g in range(w // 128):
        y = jnp.take_along_axis(v[:, g * 128:(g + 1) * 128], perm, axis=1)
        ev.append(y[:, :64])
        od.append(y[:, 64:])
    return jnp.concatenate(ev, axis=1), jnp.concatenate(od, axis=1)


def _body(x_hbm, o_hbm, bufe, bufo, obuf, insem, outsem, *, n_steps, h2):
    ph = _P // 2

    def dma_in(slot, step):
        r0 = step * _P
        for q in range(2):
            pltpu.make_async_copy(
                x_hbm.at[pl.ds(r0 + q * ph, ph), 0, :],
                bufe.at[slot, pl.ds(q * ph, ph), :],
                insem.at[slot, q],
            ).start()
            pltpu.make_async_copy(
                x_hbm.at[pl.ds(r0 + q * ph, ph), 1, :],
                bufo.at[slot, pl.ds(q * ph, ph), :],
                insem.at[slot, 2 + q],
            ).start()

    def wait_in(slot):
        for q in range(4):
            pltpu.make_async_copy(
                x_hbm.at[pl.ds(0, ph), 0, :],
                bufe.at[slot, pl.ds(0, ph), :],
                insem.at[slot, q],
            ).wait()

    def dma_out(slot, step):
        b = step // (h2 // _P)
        h0 = (step % (h2 // _P)) * _P
        for c in range(4):
            pltpu.make_async_copy(
                obuf.at[slot, c],
                o_hbm.at[b, c, pl.ds(h0, _P), :],
                outsem.at[slot, c],
            ).start()

    def wait_out(slot):
        for c in range(4):
            pltpu.make_async_copy(
                obuf.at[slot, c],
                o_hbm.at[0, c, pl.ds(0, _P), :],
                outsem.at[slot, c],
            ).wait()

    def compute(slot):
        e0, e1 = _deinterleave(bufe[slot])
        o0, o1 = _deinterleave(bufo[slot])
        obuf[slot, 0] = e0
        obuf[slot, 1] = e1
        obuf[slot, 2] = o0
        obuf[slot, 3] = o1

    dma_in(0, 0)

    def step_fn(s, _):
        slot = jax.lax.rem(s, 2)
        nxt = jax.lax.rem(s + 1, 2)

        @pl.when(s + 1 < n_steps)
        def _():
            dma_in(nxt, s + 1)

        wait_in(slot)

        @pl.when(s >= 2)
        def _():
            wait_out(slot)

        compute(slot)
        dma_out(slot, s)
        return ()

    jax.lax.fori_loop(0, n_steps, step_fn, ())
    wait_out(jax.lax.rem(n_steps - 2, 2))
    wait_out(jax.lax.rem(n_steps - 1, 2))


def kernel(x):
    B, C, H, W = x.shape
    H2, W2 = H // 2, W // 2
    x2 = x.reshape(B * H2, 2, W)
    n_steps = (B * H2) // _P
    import functools
    body = functools.partial(_body, n_steps=n_steps, h2=H2)
    return pl.pallas_call(
        body,
        in_specs=[pl.BlockSpec(memory_space=pltpu.MemorySpace.HBM)],
        out_specs=pl.BlockSpec(memory_space=pltpu.MemorySpace.HBM),
        out_shape=jax.ShapeDtypeStruct((B, 4 * C, H2, W2), x.dtype),
        scratch_shapes=[
            pltpu.VMEM((2, _P, W), x.dtype),
            pltpu.VMEM((2, _P, W), x.dtype),
            pltpu.VMEM((2, 4, _P, W2), x.dtype),
            pltpu.SemaphoreType.DMA((2, 4)),
            pltpu.SemaphoreType.DMA((2, 4)),
        ],
    )(x2)


# speedup vs baseline: 1.1251x; 1.1251x over previous
import functools

import jax
import jax.numpy as jnp
from jax.experimental import pallas as pl
from jax.experimental.pallas import tpu as pltpu

_P = 128
_SLOTS = 3
_QI = 4
_QO = 2


def _deinterleave(v):
    p, w = v.shape
    i = jax.lax.broadcasted_iota(jnp.int32, (p, 128), 1)
    perm = jnp.where(i < 64, 2 * i, 2 * i - 127)
    ev, od = [], []
    for g in range(w // 128):
        y = jnp.take_along_axis(v[:, g * 128:(g + 1) * 128], perm, axis=1)
        ev.append(y[:, :64])
        od.append(y[:, 64:])
    return jnp.concatenate(ev, axis=1), jnp.concatenate(od, axis=1)


def _body(x_hbm, o_hbm, bufe, bufo, obuf, insem, outsem, *, n_steps, h2):
    pq = _P // _QI
    po = _P // _QO

    def dma_in(slot, step):
        r0 = step * _P
        for q in range(_QI):
            pltpu.make_async_copy(
                x_hbm.at[pl.ds(r0 + q * pq, pq), 0, :],
                bufe.at[slot, pl.ds(q * pq, pq), :],
                insem.at[slot, q],
            ).start()
            pltpu.make_async_copy(
                x_hbm.at[pl.ds(r0 + q * pq, pq), 1, :],
                bufo.at[slot, pl.ds(q * pq, pq), :],
                insem.at[slot, _QI + q],
            ).start()

    def wait_in(slot):
        for q in range(2 * _QI):
            pltpu.make_async_copy(
                x_hbm.at[pl.ds(0, pq), 0, :],
                bufe.at[slot, pl.ds(0, pq), :],
                insem.at[slot, q],
            ).wait()

    def dma_out(slot, step):
        b = step // (h2 // _P)
        h0 = (step % (h2 // _P)) * _P
        for c in range(4):
            for q in range(_QO):
                pltpu.make_async_copy(
                    obuf.at[slot, c, pl.ds(q * po, po), :],
                    o_hbm.at[b, c, pl.ds(h0 + q * po, po), :],
                    outsem.at[slot, _QO * c + q],
                ).start()

    def wait_out(slot):
        for q in range(4 * _QO):
            pltpu.make_async_copy(
                obuf.at[slot, 0, pl.ds(0, po), :],
                o_hbm.at[0, 0, pl.ds(0, po), :],
                outsem.at[slot, q],
            ).wait()

    def compute(slot):
        e0, e1 = _deinterleave(bufe[slot])
        o0, o1 = _deinterleave(bufo[slot])
        obuf[slot, 0] = e0
        obuf[slot, 1] = e1
        obuf[slot, 2] = o0
        obuf[slot, 3] = o1

    for s0 in range(_SLOTS - 1):
        dma_in(s0, s0)

    def step_fn(s, _):
        slot = jax.lax.rem(s, _SLOTS)
        nxt = jax.lax.rem(s + _SLOTS - 1, _SLOTS)

        @pl.when(s + _SLOTS - 1 < n_steps)
        def _():
            dma_in(nxt, s + _SLOTS - 1)

        wait_in(slot)

        @pl.when(s >= _SLOTS)
        def _():
            wait_out(slot)

        compute(slot)
        dma_out(slot, s)
        return ()

    jax.lax.fori_loop(0, n_steps, step_fn, ())
    for s0 in range(_SLOTS):
        wait_out(jax.lax.rem(n_steps - _SLOTS + s0, _SLOTS))


def kernel(x):
    B, C, H, W = x.shape
    H2, W2 = H // 2, W // 2
    x2 = x.reshape(B * H2, 2, W)
    n_steps = (B * H2) // _P
    body = functools.partial(_body, n_steps=n_steps, h2=H2)
    return pl.pallas_call(
        body,
        in_specs=[pl.BlockSpec(memory_space=pltpu.MemorySpace.HBM)],
        out_specs=pl.BlockSpec(memory_space=pltpu.MemorySpace.HBM),
        out_shape=jax.ShapeDtypeStruct((B, 4 * C, H2, W2), x.dtype),
        scratch_shapes=[
            pltpu.VMEM((_SLOTS, _P, W), x.dtype),
            pltpu.VMEM((_SLOTS, _P, W), x.dtype),
            pltpu.VMEM((_SLOTS, 4, _P, W2), x.dtype),
            pltpu.SemaphoreType.DMA((_SLOTS, 2 * _QI)),
            pltpu.SemaphoreType.DMA((_SLOTS, 4 * _QO)),
        ],
    )(x2)


# speedup vs baseline: 1.1406x; 1.0138x over previous
import functools

import jax
import jax.numpy as jnp
from jax.experimental import pallas as pl
from jax.experimental.pallas import tpu as pltpu

_P = 128
_SLOTS = 4
_QI = 4
_QO = 2


def _deinterleave(v):
    p, w = v.shape
    i = jax.lax.broadcasted_iota(jnp.int32, (p, 128), 1)
    perm = jnp.where(i < 64, 2 * i, 2 * i - 127)
    ev, od = [], []
    for g in range(w // 128):
        y = jnp.take_along_axis(v[:, g * 128:(g + 1) * 128], perm, axis=1)
        ev.append(y[:, :64])
        od.append(y[:, 64:])
    return jnp.concatenate(ev, axis=1), jnp.concatenate(od, axis=1)


def _body(x_hbm, o_hbm, bufe, bufo, obuf, insem, outsem, *, n_steps, h2):
    pq = _P // _QI
    po = _P // _QO

    def dma_in(slot, step):
        r0 = step * _P
        for q in range(_QI):
            pltpu.make_async_copy(
                x_hbm.at[pl.ds(r0 + q * pq, pq), 0, :],
                bufe.at[slot, pl.ds(q * pq, pq), :],
                insem.at[slot],
            ).start()
            pltpu.make_async_copy(
                x_hbm.at[pl.ds(r0 + q * pq, pq), 1, :],
                bufo.at[slot, pl.ds(q * pq, pq), :],
                insem.at[slot],
            ).start()

    def wait_in(slot):
        pltpu.make_async_copy(
            x_hbm.at[pl.ds(0, _P), 0, :],
            bufe.at[slot],
            insem.at[slot],
        ).wait()
        pltpu.make_async_copy(
            x_hbm.at[pl.ds(0, _P), 1, :],
            bufo.at[slot],
            insem.at[slot],
        ).wait()

    def dma_out(slot, step):
        b = step // (h2 // _P)
        h0 = (step % (h2 // _P)) * _P
        for c in range(4):
            for q in range(_QO):
                pltpu.make_async_copy(
                    obuf.at[slot, c, pl.ds(q * po, po), :],
                    o_hbm.at[b, c, pl.ds(h0 + q * po, po), :],
                    outsem.at[slot],
                ).start()

    def wait_out(slot):
        pltpu.make_async_copy(
            obuf.at[slot],
            o_hbm.at[0, :, pl.ds(0, _P), :],
            outsem.at[slot],
        ).wait()

    def compute(slot):
        e0, e1 = _deinterleave(bufe[slot])
        o0, o1 = _deinterleave(bufo[slot])
        obuf[slot, 0] = e0
        obuf[slot, 1] = e1
        obuf[slot, 2] = o0
        obuf[slot, 3] = o1

    for s0 in range(_SLOTS - 1):
        dma_in(s0, s0)

    def step_fn(s, _):
        slot = jax.lax.rem(s, _SLOTS)
        nxt = jax.lax.rem(s + _SLOTS - 1, _SLOTS)

        @pl.when(s + _SLOTS - 1 < n_steps)
        def _():
            dma_in(nxt, s + _SLOTS - 1)

        wait_in(slot)

        @pl.when(s >= _SLOTS)
        def _():
            wait_out(slot)

        compute(slot)
        dma_out(slot, s)
        return ()

    jax.lax.fori_loop(0, n_steps, step_fn, ())
    for s0 in range(_SLOTS):
        wait_out(jax.lax.rem(n_steps - _SLOTS + s0, _SLOTS))


def kernel(x):
    B, C, H, W = x.shape
    H2, W2 = H // 2, W // 2
    x2 = x.reshape(B * H2, 2, W)
    n_steps = (B * H2) // _P
    body = functools.partial(_body, n_steps=n_steps, h2=H2)
    return pl.pallas_call(
        body,
        in_specs=[pl.BlockSpec(memory_space=pltpu.MemorySpace.HBM)],
        out_specs=pl.BlockSpec(memory_space=pltpu.MemorySpace.HBM),
        out_shape=jax.ShapeDtypeStruct((B, 4 * C, H2, W2), x.dtype),
        scratch_shapes=[
            pltpu.VMEM((_SLOTS, _P, W), x.dtype),
            pltpu.VMEM((_SLOTS, _P, W), x.dtype),
            pltpu.VMEM((_SLOTS, 4, _P, W2), x.dtype),
            pltpu.SemaphoreType.DMA((_SLOTS,)),
            pltpu.SemaphoreType.DMA((_SLOTS,)),
        ],
    )(x2)
